# Initial kernel scaffold; baseline (speedup 1.0000x reference)
import jax
import jax.numpy as jnp
from jax.experimental import pallas as pl


def kernel(x, DV2_H, invDE_HT_DV2, W, W1, b1, W2, b2, bn1_gamma, bn1_beta, bn2_gamma, bn2_beta):
    raise NotImplementedError("write your pallas kernel here")


# speedup vs baseline: 2.7955x; 2.7955x over previous
import jax
import jax.numpy as jnp
from jax.experimental import pallas as pl

_EPS = 1e-5
_N_CLASS = 40


def _fused_hgnn_kernel(x_ref, dvh_ref, inv_ref, w_ref, w1_ref, b1_ref,
                       w2_ref, b2_ref, g1_ref, be1_ref, g2_ref, be2_ref,
                       out_ref):
    f32 = jnp.float32
    x = x_ref[...]

    mu1 = jnp.mean(x, axis=0, keepdims=True)
    var1 = jnp.mean((x - mu1) ** 2, axis=0, keepdims=True)
    scale1 = g1_ref[...] * jax.lax.rsqrt(var1 + _EPS)
    xbn = x * scale1 + (be1_ref[...] - scale1 * mu1)

    h1 = jnp.dot(xbn, w1_ref[...], preferred_element_type=f32) + b1_ref[...]

    aw = dvh_ref[...] * w_ref[...]
    t = jnp.dot(inv_ref[...], h1, preferred_element_type=f32)
    h = jnp.dot(aw, t, preferred_element_type=f32)

    mu2 = jnp.mean(h, axis=0, keepdims=True)
    var2 = jnp.mean((h - mu2) ** 2, axis=0, keepdims=True)
    scale2 = g2_ref[...] * jax.lax.rsqrt(var2 + _EPS)
    r = jnp.maximum(h * scale2 + (be2_ref[...] - scale2 * mu2), 0.0)

    mu3 = jnp.mean(r, axis=0, keepdims=True)
    var3 = jnp.mean((r - mu3) ** 2, axis=0, keepdims=True)
    scale3 = g2_ref[...] * jax.lax.rsqrt(var3 + _EPS)
    r2 = r * scale3 + (be2_ref[...] - scale3 * mu3)

    u = jnp.dot(r2, w2_ref[...], preferred_element_type=f32) + b2_ref[...]
    t2 = jnp.dot(inv_ref[...], u, preferred_element_type=f32)
    out_ref[...] = jnp.dot(aw, t2, preferred_element_type=f32)


def kernel(x, DV2_H, invDE_HT_DV2, W, W1, b1, W2, b2,
           bn1_gamma, bn1_beta, bn2_gamma, bn2_beta):
    n, in_ch = x.shape
    m = DV2_H.shape[1]
    n_hid = W1.shape[1]
    c_pad = 128

    W2p = jnp.zeros((n_hid, c_pad), dtype=W2.dtype).at[:, :_N_CLASS].set(W2)
    b2p = jnp.zeros((1, c_pad), dtype=b2.dtype).at[0, :_N_CLASS].set(b2)

    out = pl.pallas_call(
        _fused_hgnn_kernel,
        out_shape=jax.ShapeDtypeStruct((n, c_pad), jnp.float32),
    )(
        x, DV2_H, invDE_HT_DV2,
        W.reshape(1, m), W1, b1.reshape(1, n_hid),
        W2p, b2p,
        bn1_gamma.reshape(1, in_ch), bn1_beta.reshape(1, in_ch),
        bn2_gamma.reshape(1, n_hid), bn2_beta.reshape(1, n_hid),
    )
    return out[:, :_N_CLASS]


# speedup vs baseline: 2.9524x; 1.0561x over previous
import jax
import jax.numpy as jnp
from jax.experimental import pallas as pl
from jax.experimental.pallas import tpu as pltpu

_EPS = 1e-5
_N_CLASS = 40


def _fused_hgnn_kernel(x_ref, dvh_hbm, inv_hbm, w_ref, w1_ref, b1_ref,
                       w2_ref, b2_ref, g1_ref, be1_ref, g2_ref, be2_ref,
                       out_ref, dvh_ref, inv_ref, sem_dvh, sem_inv):
    f32 = jnp.float32
    cp_inv = pltpu.make_async_copy(inv_hbm, inv_ref, sem_inv)
    cp_dvh = pltpu.make_async_copy(dvh_hbm, dvh_ref, sem_dvh)
    cp_inv.start()
    cp_dvh.start()

    x = x_ref[...]

    mu1 = jnp.mean(x, axis=0, keepdims=True)
    var1 = jnp.mean((x - mu1) ** 2, axis=0, keepdims=True)
    scale1 = g1_ref[...] * jax.lax.rsqrt(var1 + _EPS)
    xbn = x * scale1 + (be1_ref[...] - scale1 * mu1)

    h1 = jnp.dot(xbn, w1_ref[...], preferred_element_type=f32) + b1_ref[...]

    cp_inv.wait()
    t = jnp.dot(inv_ref[...], h1, preferred_element_type=f32)
    cp_dvh.wait()
    aw = dvh_ref[...] * w_ref[...]
    h = jnp.dot(aw, t, preferred_element_type=f32)

    mu2 = jnp.mean(h, axis=0, keepdims=True)
    var2 = jnp.mean((h - mu2) ** 2, axis=0, keepdims=True)
    scale2 = g2_ref[...] * jax.lax.rsqrt(var2 + _EPS)
    r = jnp.maximum(h * scale2 + (be2_ref[...] - scale2 * mu2), 0.0)

    mu3 = jnp.mean(r, axis=0, keepdims=True)
    var3 = jnp.mean((r - mu3) ** 2, axis=0, keepdims=True)
    scale3 = g2_ref[...] * jax.lax.rsqrt(var3 + _EPS)
    r2 = r * scale3 + (be2_ref[...] - scale3 * mu3)

    u = jnp.dot(r2, w2_ref[...], preferred_element_type=f32) + b2_ref[...]
    t2 = jnp.dot(inv_ref[...], u, preferred_element_type=f32)
    out_ref[...] = jnp.dot(aw, t2, preferred_element_type=f32)


def kernel(x, DV2_H, invDE_HT_DV2, W, W1, b1, W2, b2,
           bn1_gamma, bn1_beta, bn2_gamma, bn2_beta):
    n, in_ch = x.shape
    m = DV2_H.shape[1]
    n_hid = W1.shape[1]
    c_pad = 128

    W2p = jnp.zeros((n_hid, c_pad), dtype=W2.dtype).at[:, :_N_CLASS].set(W2)
    b2p = jnp.zeros((1, c_pad), dtype=b2.dtype).at[0, :_N_CLASS].set(b2)

    vmem = pl.BlockSpec(memory_space=pltpu.MemorySpace.VMEM)
    hbm = pl.BlockSpec(memory_space=pl.ANY)
    out = pl.pallas_call(
        _fused_hgnn_kernel,
        out_shape=jax.ShapeDtypeStruct((n, c_pad), jnp.float32),
        in_specs=[vmem, hbm, hbm] + [vmem] * 9,
        out_specs=vmem,
        scratch_shapes=[
            pltpu.VMEM((n, m), jnp.float32),
            pltpu.VMEM((m, n), jnp.float32),
            pltpu.SemaphoreType.DMA,
            pltpu.SemaphoreType.DMA,
        ],
    )(
        x, DV2_H, invDE_HT_DV2,
        W.reshape(1, m), W1, b1.reshape(1, n_hid),
        W2p, b2p,
        bn1_gamma.reshape(1, in_ch), bn1_beta.reshape(1, in_ch),
        bn2_gamma.reshape(1, n_hid), bn2_beta.reshape(1, n_hid),
    )
    return out[:, :_N_CLASS]


# speedup vs baseline: 2.9609x; 1.0029x over previous
import jax
import jax.numpy as jnp
from jax.experimental import pallas as pl
from jax.experimental.pallas import tpu as pltpu

_EPS = 1e-5
_N_CLASS = 40


def _fused_hgnn_kernel(x_ref, dvh_hbm, inv_hbm, w_ref, w1_ref, b1_ref,
                       w2_ref, b2_ref, g1_ref, be1_ref, g2_ref, be2_ref,
                       out_ref, dvh_ref, inv_ref, sem_dvh, sem_inv):
    f32 = jnp.float32
    cp_inv = pltpu.make_async_copy(inv_hbm, inv_ref, sem_inv)
    cp_dvh = pltpu.make_async_copy(dvh_hbm, dvh_ref, sem_dvh)
    cp_inv.start()
    cp_dvh.start()

    x = x_ref[...]

    mu1 = jnp.mean(x, axis=0, keepdims=True)
    var1 = jnp.mean((x - mu1) ** 2, axis=0, keepdims=True)
    scale1 = g1_ref[...] * jax.lax.rsqrt(var1 + _EPS)
    xbn = x * scale1 + (be1_ref[...] - scale1 * mu1)

    h1 = jnp.dot(xbn, w1_ref[...], preferred_element_type=f32) + b1_ref[...]

    cp_inv.wait()
    t = jnp.dot(inv_ref[...], h1, preferred_element_type=f32)
    cp_dvh.wait()
    aw = dvh_ref[...] * w_ref[...]

    awc = aw - jnp.mean(aw, axis=0, keepdims=True)
    hc = jnp.dot(awc, t, preferred_element_type=f32)
    var2 = jnp.mean(hc * hc, axis=0, keepdims=True)
    scale2 = g2_ref[...] * jax.lax.rsqrt(var2 + _EPS)
    r = jnp.maximum(hc * scale2 + be2_ref[...], 0.0)

    mu3 = jnp.mean(r, axis=0, keepdims=True)
    var3 = jnp.mean((r - mu3) ** 2, axis=0, keepdims=True)
    scale3 = g2_ref[...] * jax.lax.rsqrt(var3 + _EPS)
    r2 = r * scale3 + (be2_ref[...] - scale3 * mu3)

    u = jnp.dot(r2, w2_ref[...], preferred_element_type=f32) + b2_ref[...]
    t2 = jnp.dot(inv_ref[...], u, preferred_element_type=f32)
    out_ref[...] = jnp.dot(aw, t2, preferred_element_type=f32)


def kernel(x, DV2_H, invDE_HT_DV2, W, W1, b1, W2, b2,
           bn1_gamma, bn1_beta, bn2_gamma, bn2_beta):
    n, in_ch = x.shape
    m = DV2_H.shape[1]
    n_hid = W1.shape[1]
    c_pad = 128

    W2p = jnp.zeros((n_hid, c_pad), dtype=W2.dtype).at[:, :_N_CLASS].set(W2)
    b2p = jnp.zeros((1, c_pad), dtype=b2.dtype).at[0, :_N_CLASS].set(b2)

    vmem = pl.BlockSpec(memory_space=pltpu.MemorySpace.VMEM)
    hbm = pl.BlockSpec(memory_space=pl.ANY)
    out = pl.pallas_call(
        _fused_hgnn_kernel,
        out_shape=jax.ShapeDtypeStruct((n, c_pad), jnp.float32),
        in_specs=[vmem, hbm, hbm] + [vmem] * 9,
        out_specs=vmem,
        scratch_shapes=[
            pltpu.VMEM((n, m), jnp.float32),
            pltpu.VMEM((m, n), jnp.float32),
            pltpu.SemaphoreType.DMA,
            pltpu.SemaphoreType.DMA,
        ],
    )(
        x, DV2_H, invDE_HT_DV2,
        W.reshape(1, m), W1, b1.reshape(1, n_hid),
        W2p, b2p,
        bn1_gamma.reshape(1, in_ch), bn1_beta.reshape(1, in_ch),
        bn2_gamma.reshape(1, n_hid), bn2_beta.reshape(1, n_hid),
    )
    return out[:, :_N_CLASS]
